# baseline (speedup 1.0000x reference)
import functools

import jax
import jax.numpy as jnp
from jax import lax
from jax.experimental import pallas as pl
from jax.experimental.pallas import tpu as pltpu
from jax.experimental.pallas import tpu_sc as plsc

VOCAB = 1000000
HIDDEN = 64
BATCH = 16384
HIST = 50

B = BATCH * HIST
NC, NS = 2, 16
NW = NC * NS
BPW = B // NW
CHUNK = 128
NCHUNK = BPW // CHUNK
NBUF = 4

_mesh = plsc.VectorSubcoreMesh(core_axis_name="c", subcore_axis_name="s")


@functools.partial(
    pl.kernel,
    mesh=_mesh,
    out_type=jax.ShapeDtypeStruct((B, HIDDEN), jnp.float32),
    scratch_types=[
        pltpu.VMEM((BPW,), jnp.int32),
        [pltpu.VMEM((CHUNK, HIDDEN), jnp.float32) for _ in range(NBUF)],
        [pltpu.SemaphoreType.DMA for _ in range(NBUF)],
        [pltpu.SemaphoreType.DMA for _ in range(NBUF)],
    ],
    compiler_params=pltpu.CompilerParams(use_tc_tiling_on_sc=False),
)
def _gather_kernel(idx_hbm, table_hbm, out_hbm, idx_v, rows, sem_in, sem_out):
    wid = lax.axis_index("s") * NC + lax.axis_index("c")
    base = wid * BPW

    def fire(g, j):
        pltpu.async_copy(
            table_hbm.at[idx_v.at[pl.ds(g * CHUNK, CHUNK)]], rows[j], sem_in[j]
        )

    def wait_gather(j):
        pltpu.make_async_copy(
            table_hbm.at[idx_v.at[pl.ds(0, CHUNK)]], rows[j], sem_in[j]
        ).wait()

    def start_writeback(g, j):
        pltpu.async_copy(
            rows[j], out_hbm.at[pl.ds(base + g * CHUNK, CHUNK)], sem_out[j]
        )

    def wait_writeback(j):
        pltpu.make_async_copy(
            rows[j], out_hbm.at[pl.ds(base, CHUNK)], sem_out[j]
        ).wait()

    pltpu.sync_copy(idx_hbm.at[pl.ds(base, BPW)], idx_v)

    fire(0, 0)
    fire(1, 1)

    for j in range(NBUF):
        wait_gather(j)
        start_writeback(j, j)
        if j + 2 < NBUF:
            fire(j + 2, j + 2)
        else:
            wait_writeback((j + 2) % NBUF)
            fire(j + 2, (j + 2) % NBUF)

    def body(go, carry):
        for j in range(NBUF):
            g = go * NBUF + j
            wait_gather(j)
            start_writeback(g, j)
            wait_writeback((j + 2) % NBUF)
            fire(g + 2, (j + 2) % NBUF)
        return carry

    lax.fori_loop(1, NCHUNK // NBUF - 1, body, 0)

    for j in range(NBUF):
        g = NCHUNK - NBUF + j
        wait_gather(j)
        start_writeback(g, j)
        if j < 2:
            wait_writeback((j + 2) % NBUF)
            fire(g + 2, (j + 2) % NBUF)

    for j in range(NBUF):
        wait_writeback(j)


def kernel(input, weight):
    idx = input.T.reshape(-1)
    out = _gather_kernel(idx, weight)
    return out.reshape(HIST, BATCH, HIDDEN).transpose(1, 0, 2)
